# baseline (speedup 1.0000x reference)
import functools

import jax
import jax.numpy as jnp
from jax import lax
from jax.experimental import pallas as pl
from jax.experimental.pallas import tpu as pltpu
from jax.experimental.pallas import tpu_sc as plsc

_EPS = 1e-30
_LN2 = 0.6931471805599453
_B = 16384
_T = 32
_L = 16


def _log16(x):
    bits = lax.bitcast_convert_type(x, jnp.int32)
    e = (bits >> 23) - 127
    m = lax.bitcast_convert_type((bits & 0x007FFFFF) | 0x3F800000, jnp.float32)
    big = m > 1.4142135381698608
    m = jnp.where(big, m * 0.5, m)
    ef = e.astype(jnp.float32) + jnp.where(big, 1.0, 0.0)
    s = (m - 1.0) / (m + 1.0)
    z = s * s
    poly = 2.0 + z * (0.6666666666 + z * (0.4 + z * 0.2857142857))
    return ef * _LN2 + s * poly


def _make_sc_kernel():
    info = plsc.get_sparse_core_info()
    nc, ns = info.num_cores, info.num_subcores
    nw = nc * ns
    bpw = _B // nw
    mesh = plsc.VectorSubcoreMesh(core_axis_name="c", subcore_axis_name="s")

    @functools.partial(
        pl.kernel,
        mesh=mesh,
        out_type=jax.ShapeDtypeStruct((_B,), jnp.float32),
        compiler_params=pltpu.CompilerParams(needs_layout_passes=False),
        scratch_types=[
            pltpu.VMEM((_T,), jnp.float32),
            pltpu.VMEM((bpw,), jnp.int32),
            pltpu.VMEM((bpw,), jnp.float32),
        ],
    )
    def sc_kernel(prob_hbm, idx_hbm, out_hbm, tab_v, idx_v, out_v):
        wid = lax.axis_index("s") * nc + lax.axis_index("c")
        base = wid * bpw
        pltpu.sync_copy(prob_hbm, tab_v)
        pltpu.sync_copy(idx_hbm.at[pl.ds(base, bpw)], idx_v)
        for j in range(_T // _L):
            x = tab_v[pl.ds(j * _L, _L)] + _EPS
            tab_v[pl.ds(j * _L, _L)] = _log16(x)
        for i in range(bpw // _L):
            idx = idx_v[pl.ds(i * _L, _L)]
            out_v[pl.ds(i * _L, _L)] = plsc.load_gather(tab_v, [idx])
        pltpu.sync_copy(out_v, out_hbm.at[pl.ds(base, bpw)])

    return sc_kernel


_SC_KERNEL = _make_sc_kernel()


def kernel(batch_n_nodes, prob):
    idx = batch_n_nodes.astype(jnp.int32)
    prob_padded = jnp.concatenate(
        [prob.astype(jnp.float32), jnp.ones((_T - prob.shape[0],), jnp.float32)]
    )
    return _SC_KERNEL(prob_padded, idx)


# speedup vs baseline: 1.0460x; 1.0460x over previous
import functools

import jax
import jax.numpy as jnp
from jax import lax
from jax.experimental import pallas as pl
from jax.experimental.pallas import tpu as pltpu
from jax.experimental.pallas import tpu_sc as plsc

_EPS = 1e-30
_LN2 = 0.6931471805599453
_B = 16384
_V = 29
_T = 32
_L = 16


def _log16(x):
    bits = lax.bitcast_convert_type(x, jnp.int32)
    e = (bits >> 23) - 127
    m = lax.bitcast_convert_type((bits & 0x007FFFFF) | 0x3F800000, jnp.float32)
    big = m > 1.4142135381698608
    m = jnp.where(big, m * 0.5, m)
    ef = e.astype(jnp.float32) + jnp.where(big, 1.0, 0.0)
    s = (m - 1.0) / (m + 1.0)
    z = s * s
    poly = 2.0 + z * (0.6666666666 + z * (0.4 + z * 0.2857142857))
    return ef * _LN2 + s * poly


def _make_sc_kernel():
    info = plsc.get_sparse_core_info()
    nc, ns = info.num_cores, info.num_subcores
    nw = nc * ns
    bpw = _B // nw
    mesh = plsc.VectorSubcoreMesh(core_axis_name="c", subcore_axis_name="s")

    @functools.partial(
        pl.kernel,
        mesh=mesh,
        out_type=jax.ShapeDtypeStruct((_B,), jnp.float32),
        compiler_params=pltpu.CompilerParams(needs_layout_passes=False),
        scratch_types=[
            pltpu.VMEM((_T,), jnp.float32),
            pltpu.VMEM((bpw,), jnp.int32),
            pltpu.VMEM((bpw,), jnp.float32),
            pltpu.SemaphoreType.DMA,
        ],
    )
    def sc_kernel(prob_hbm, idx_hbm, out_hbm, tab_v, idx_v, out_v, sem):
        wid = lax.axis_index("s") * nc + lax.axis_index("c")
        base = wid * bpw
        idx_cp = pltpu.async_copy(idx_hbm.at[pl.ds(base, bpw)], idx_v, sem)
        pltpu.sync_copy(prob_hbm, tab_v.at[pl.ds(0, _V)])
        for j in range(_T // _L):
            x = tab_v[pl.ds(j * _L, _L)] + _EPS
            tab_v[pl.ds(j * _L, _L)] = _log16(x)
        idx_cp.wait()
        for i in range(bpw // _L):
            idx = idx_v[pl.ds(i * _L, _L)]
            out_v[pl.ds(i * _L, _L)] = plsc.load_gather(tab_v, [idx])
        pltpu.sync_copy(out_v, out_hbm.at[pl.ds(base, bpw)])

    return sc_kernel


_SC_KERNEL = _make_sc_kernel()


def kernel(batch_n_nodes, prob):
    return _SC_KERNEL(prob.astype(jnp.float32), batch_n_nodes.astype(jnp.int32))


# speedup vs baseline: 1.0727x; 1.0256x over previous
import functools

import jax
import jax.numpy as jnp
from jax import lax
from jax.experimental import pallas as pl
from jax.experimental.pallas import tpu as pltpu
from jax.experimental.pallas import tpu_sc as plsc

_EPS = 1e-30
_LN2 = 0.6931471805599453
_B = 16384
_V = 29
_T = 32
_L = 16


def _log16(x):
    bits = lax.bitcast_convert_type(x, jnp.int32)
    e = (bits >> 23) - 127
    m = lax.bitcast_convert_type((bits & 0x007FFFFF) | 0x3F800000, jnp.float32)
    big = m > 1.4142135381698608
    m = jnp.where(big, m * 0.5, m)
    ef = e.astype(jnp.float32) + jnp.where(big, 1.0, 0.0)
    s = (m - 1.0) / (m + 1.0)
    z = s * s
    poly = 2.0 + z * (0.6666666666 + z * (0.4 + z * 0.2857142857))
    return ef * _LN2 + s * poly


def _make_sc_kernel():
    info = plsc.get_sparse_core_info()
    nc, ns = info.num_cores, info.num_subcores
    nw = nc * ns
    bpw = _B // nw
    mesh = plsc.VectorSubcoreMesh(core_axis_name="c", subcore_axis_name="s")

    @functools.partial(
        pl.kernel,
        mesh=mesh,
        out_type=jax.ShapeDtypeStruct((_B,), jnp.float32),
        compiler_params=pltpu.CompilerParams(needs_layout_passes=False),
        scratch_types=[
            pltpu.VMEM((_T,), jnp.float32),
            pltpu.VMEM((bpw,), jnp.int32),
            pltpu.VMEM((bpw,), jnp.float32),
            pltpu.SemaphoreType.DMA,
            pltpu.SemaphoreType.DMA,
        ],
    )
    def sc_kernel(prob_hbm, idx_hbm, out_hbm, tab_v, idx_v, out_v, sem, osem):
        wid = lax.axis_index("s") * nc + lax.axis_index("c")
        base = wid * bpw
        idx_cp = pltpu.async_copy(idx_hbm.at[pl.ds(base, bpw)], idx_v, sem)
        pltpu.sync_copy(prob_hbm, tab_v.at[pl.ds(0, _V)])
        for j in range(_T // _L):
            x = tab_v[pl.ds(j * _L, _L)] + _EPS
            tab_v[pl.ds(j * _L, _L)] = _log16(x)
        idx_cp.wait()
        half = bpw // 2
        out_cps = []
        for h in range(2):
            for i in range(h * half // _L, (h + 1) * half // _L):
                idx = idx_v[pl.ds(i * _L, _L)]
                out_v[pl.ds(i * _L, _L)] = plsc.load_gather(tab_v, [idx])
            out_cps.append(pltpu.async_copy(
                out_v.at[pl.ds(h * half, half)],
                out_hbm.at[pl.ds(base + h * half, half)], osem))
        for cp in out_cps:
            cp.wait()

    return sc_kernel


_SC_KERNEL = _make_sc_kernel()


def kernel(batch_n_nodes, prob):
    return _SC_KERNEL(prob.astype(jnp.float32), batch_n_nodes.astype(jnp.int32))
